# Initial kernel scaffold; baseline (speedup 1.0000x reference)
import functools
import math

import jax
import jax.numpy as jnp
from jax import lax
from jax.experimental import pallas as pl
from jax.experimental.pallas import tpu as pltpu
from jax.experimental.pallas import tpu_sc as plsc

W_S = 4
N_W = W_S * W_S
B, L, D = 32, 577, 768
CLS = L // 2
H = 24
HW = 6
L_PAD = 640
CHUNK = 64
N_CHUNK = 9


def _build_rot(r, epsilon=1e-08):
    norms = jnp.linalg.norm(r, axis=1, keepdims=True)
    r = r / (norms + epsilon)
    angles = jnp.arctan2(r[:, 0], r[:, 1])
    cos = jnp.cos(angles)
    sin = jnp.sin(angles)
    row0 = jnp.stack([cos, -sin], axis=-1)
    row1 = jnp.stack([sin, cos], axis=-1)
    return jnp.stack([row0, row1], axis=1)


def _row_tables(scale, rotation, mean_p):
    scale_e = jnp.exp(scale)
    left = jax.vmap(jnp.diag)(scale_e)
    right = _build_rot(rotation)
    transform = left @ right
    cov = transform @ jnp.swapaxes(transform, -2, -1)
    chol = jnp.linalg.cholesky(cov)
    inv_cov = jax.vmap(
        lambda c: jax.scipy.linalg.cho_solve((c, True), jnp.eye(2, dtype=c.dtype))
    )(chol)
    grid_y, grid_x = jnp.meshgrid(
        jnp.arange(HW, dtype=jnp.float32),
        jnp.arange(HW, dtype=jnp.float32),
        indexing="ij",
    )
    grid = jnp.stack([grid_x, grid_y], axis=-1)
    mean = jnp.exp(mean_p)
    mean_mean = jnp.mean(mean, axis=1, keepdims=True)
    mean_std = jnp.std(mean, axis=1, keepdims=True, ddof=1)
    mean = (mean - mean_mean) / (mean_std + 1e-05)
    mean = mean * (HW // 2) + HW // 2
    mean = jnp.clip(mean, 0.0, float(HW // 2))
    diff = grid[None, :, :, :] - mean[:, None, None, :]
    maha = jnp.einsum("nhwi,nij,nhwj->nhw", diff, inv_cov, diff)
    weights = jax.nn.sigmoid(jnp.exp(-0.5 * maha)).reshape(N_W, HW * HW)
    idx = jnp.argsort(-weights, axis=1)
    sorted_w = jnp.take_along_axis(weights, idx, axis=1)

    n = jnp.arange(N_W)[:, None]
    j = jnp.arange(HW * HW)[None, :]
    wy, wx = n // W_S, n % W_S
    iy, ix = j // HW, j % HW
    ro = (wy * HW + iy) * H + wx * HW + ix
    sy, sx = idx // HW, idx % HW
    rs = (wy * HW + sy) * H + wx * HW + sx
    srow = jnp.zeros((H * H,), jnp.int32).at[ro.ravel()].set(rs.ravel().astype(jnp.int32))
    wrow = jnp.zeros((H * H,), jnp.float32).at[ro.ravel()].set(sorted_w.ravel())
    src_sp = srow + (srow >= CLS).astype(jnp.int32)
    src = jnp.concatenate(
        [src_sp[:CLS], jnp.array([CLS], jnp.int32), src_sp[CLS:]]
    )
    wgt = jnp.concatenate(
        [wrow[:CLS], jnp.array([1.0], jnp.float32), wrow[CLS:]]
    )
    src = jnp.concatenate([src, jnp.zeros((L_PAD - L,), jnp.int32)])
    wgt = jnp.concatenate([wgt, jnp.zeros((L_PAD - L,), jnp.float32)])
    wgt16 = jnp.broadcast_to(wgt[:, None], (L_PAD, 16))
    return src, wgt16


def _sc_body(x_hbm, src_hbm, wgt_hbm, out_hbm, src_v, wgt_v, rows_v, tail_v, sem):
    info = plsc.get_sparse_core_info()
    nc = info.num_cores
    wid = lax.axis_index("s") * nc + lax.axis_index("c")
    base = wid * L

    pltpu.sync_copy(src_hbm, src_v)
    pltpu.sync_copy(wgt_hbm, wgt_v)
    for k in range(L_PAD // 16):
        sl = pl.ds(k * 16, 16)
        src_v[sl] = src_v[sl] + base

    def scale_rows(buf, nrows, row0):
        def body(r, _):
            wv = wgt_v[row0 + r, :]
            for c in range(D // 16):
                cs = pl.ds(c * 16, 16)
                buf[r, cs] = buf[r, cs] * wv
            return 0

        lax.fori_loop(0, nrows, body, 0)

    for ch in range(N_CHUNK):
        off = ch * CHUNK
        pltpu.async_copy(x_hbm.at[src_v.at[pl.ds(off, CHUNK)]], rows_v, sem).wait()
        scale_rows(rows_v, CHUNK, off)
        pltpu.sync_copy(rows_v, out_hbm.at[pl.ds(base + off, CHUNK)])
    pltpu.async_copy(x_hbm.at[src_v.at[pl.ds(N_CHUNK * CHUNK, 8)]], tail_v, sem).wait()
    scale_rows(tail_v, 1, N_CHUNK * CHUNK)
    pltpu.sync_copy(tail_v.at[pl.ds(0, 1)], out_hbm.at[pl.ds(base + N_CHUNK * CHUNK, 1)])


@jax.jit
def kernel(x, scale, rotation, mean):
    src, wgt16 = _row_tables(scale, rotation, mean)
    x2d = x.reshape(B * L, D)

    mesh = plsc.VectorSubcoreMesh(core_axis_name="c", subcore_axis_name="s")
    run = functools.partial(
        pl.kernel,
        mesh=mesh,
        out_type=jax.ShapeDtypeStruct((B * L, D), jnp.float32),
        scratch_types=[
            pltpu.VMEM((L_PAD,), jnp.int32),
            pltpu.VMEM((L_PAD, 16), jnp.float32),
            pltpu.VMEM((CHUNK, D), jnp.float32),
            pltpu.VMEM((8, D), jnp.float32),
            pltpu.SemaphoreType.DMA,
        ],
    )(_sc_body)
    out2d = run(x2d, src, wgt16)
    return out2d.reshape(B, L, D)


# speedup vs baseline: 1.7218x; 1.7218x over previous
import functools
import math

import jax
import jax.numpy as jnp
from jax import lax
from jax.experimental import pallas as pl
from jax.experimental.pallas import tpu as pltpu
from jax.experimental.pallas import tpu_sc as plsc

W_S = 4
N_W = W_S * W_S
B, L, D = 32, 577, 768
CLS = L // 2
H = 24
HW = 6
L_PAD = 640
CHUNK = 32
N_CHUNK = 18


def _build_rot(r, epsilon=1e-08):
    norms = jnp.linalg.norm(r, axis=1, keepdims=True)
    r = r / (norms + epsilon)
    angles = jnp.arctan2(r[:, 0], r[:, 1])
    cos = jnp.cos(angles)
    sin = jnp.sin(angles)
    row0 = jnp.stack([cos, -sin], axis=-1)
    row1 = jnp.stack([sin, cos], axis=-1)
    return jnp.stack([row0, row1], axis=1)


def _row_tables(scale, rotation, mean_p):
    scale_e = jnp.exp(scale)
    left = jax.vmap(jnp.diag)(scale_e)
    right = _build_rot(rotation)
    transform = left @ right
    cov = transform @ jnp.swapaxes(transform, -2, -1)
    chol = jnp.linalg.cholesky(cov)
    inv_cov = jax.vmap(
        lambda c: jax.scipy.linalg.cho_solve((c, True), jnp.eye(2, dtype=c.dtype))
    )(chol)
    grid_y, grid_x = jnp.meshgrid(
        jnp.arange(HW, dtype=jnp.float32),
        jnp.arange(HW, dtype=jnp.float32),
        indexing="ij",
    )
    grid = jnp.stack([grid_x, grid_y], axis=-1)
    mean = jnp.exp(mean_p)
    mean_mean = jnp.mean(mean, axis=1, keepdims=True)
    mean_std = jnp.std(mean, axis=1, keepdims=True, ddof=1)
    mean = (mean - mean_mean) / (mean_std + 1e-05)
    mean = mean * (HW // 2) + HW // 2
    mean = jnp.clip(mean, 0.0, float(HW // 2))
    diff = grid[None, :, :, :] - mean[:, None, None, :]
    maha = jnp.einsum("nhwi,nij,nhwj->nhw", diff, inv_cov, diff)
    weights = jax.nn.sigmoid(jnp.exp(-0.5 * maha)).reshape(N_W, HW * HW)
    idx = jnp.argsort(-weights, axis=1)
    sorted_w = jnp.take_along_axis(weights, idx, axis=1)

    n = jnp.arange(N_W)[:, None]
    j = jnp.arange(HW * HW)[None, :]
    wy, wx = n // W_S, n % W_S
    iy, ix = j // HW, j % HW
    ro = (wy * HW + iy) * H + wx * HW + ix
    sy, sx = idx // HW, idx % HW
    rs = (wy * HW + sy) * H + wx * HW + sx
    srow = jnp.zeros((H * H,), jnp.int32).at[ro.ravel()].set(rs.ravel().astype(jnp.int32))
    wrow = jnp.zeros((H * H,), jnp.float32).at[ro.ravel()].set(sorted_w.ravel())
    src_sp = srow + (srow >= CLS).astype(jnp.int32)
    src = jnp.concatenate(
        [src_sp[:CLS], jnp.array([CLS], jnp.int32), src_sp[CLS:]]
    )
    wgt = jnp.concatenate(
        [wrow[:CLS], jnp.array([1.0], jnp.float32), wrow[CLS:]]
    )
    src = jnp.concatenate([src, jnp.zeros((L_PAD - L,), jnp.int32)])
    wgt = jnp.concatenate([wgt, jnp.zeros((L_PAD - L,), jnp.float32)])
    wgt16 = jnp.broadcast_to(wgt[:, None], (L_PAD, 16))
    return src, wgt16


def _sc_body(x_hbm, src_hbm, wgt_hbm, out_hbm, src_v, wgt_v, rows_v, sem):
    info = plsc.get_sparse_core_info()
    nc = info.num_cores
    wid = lax.axis_index("s") * nc + lax.axis_index("c")
    base = wid * L

    out_b = out_hbm.at[wid]

    pltpu.sync_copy(src_hbm, src_v)
    pltpu.sync_copy(wgt_hbm, wgt_v)
    for k in range(L_PAD // 16):
        sl = pl.ds(k * 16, 16)
        src_v[sl] = src_v[sl] + base

    def scale_rows(buf, nrows, row0):
        def body(r, _):
            wv = wgt_v[row0 + r, :]
            for c in range(D // 16):
                cs = pl.ds(c * 16, 16)
                buf[r, cs] = buf[r, cs] * wv
            return 0

        lax.fori_loop(0, nrows, body, 0)

    for ch in range(N_CHUNK):
        off = ch * CHUNK
        pltpu.async_copy(x_hbm.at[src_v.at[pl.ds(off, CHUNK)]], rows_v, sem).wait()
        scale_rows(rows_v, CHUNK, off)
        pltpu.sync_copy(rows_v, out_b.at[pl.ds(off, CHUNK)])
    tail_v = rows_v.at[pl.ds(0, 8)]
    pltpu.async_copy(x_hbm.at[src_v.at[pl.ds(N_CHUNK * CHUNK, 8)]], tail_v, sem).wait()
    scale_rows(rows_v, 1, N_CHUNK * CHUNK)
    pltpu.sync_copy(rows_v.at[pl.ds(0, 1)], out_b.at[pl.ds(N_CHUNK * CHUNK, 1)])


@jax.jit
def kernel(x, scale, rotation, mean):
    src, wgt16 = _row_tables(scale, rotation, mean)
    x2d = x.reshape(B * L, D)

    mesh = plsc.VectorSubcoreMesh(core_axis_name="c", subcore_axis_name="s")
    run = functools.partial(
        pl.kernel,
        mesh=mesh,
        out_type=jax.ShapeDtypeStruct((B, L, D), jnp.float32),
        scratch_types=[
            pltpu.VMEM((L_PAD,), jnp.int32),
            pltpu.VMEM((L_PAD, 16), jnp.float32),
            pltpu.VMEM((CHUNK, D), jnp.float32),
            pltpu.SemaphoreType.DMA,
        ],
    )(_sc_body)
    return run(x2d, src, wgt16)


# speedup vs baseline: 2.0723x; 1.2036x over previous
import functools
import math

import jax
import jax.numpy as jnp
from jax import lax
from jax.experimental import pallas as pl
from jax.experimental.pallas import tpu as pltpu
from jax.experimental.pallas import tpu_sc as plsc

W_S = 4
N_W = W_S * W_S
B, L, D = 32, 577, 768
CLS = L // 2
H = 24
HW = 6
L_PAD = 640
CHUNK = 16
N_CHUNK = 36


def _build_rot(r, epsilon=1e-08):
    norms = jnp.linalg.norm(r, axis=1, keepdims=True)
    r = r / (norms + epsilon)
    angles = jnp.arctan2(r[:, 0], r[:, 1])
    cos = jnp.cos(angles)
    sin = jnp.sin(angles)
    row0 = jnp.stack([cos, -sin], axis=-1)
    row1 = jnp.stack([sin, cos], axis=-1)
    return jnp.stack([row0, row1], axis=1)


def _row_tables(scale, rotation, mean_p):
    scale_e = jnp.exp(scale)
    left = jax.vmap(jnp.diag)(scale_e)
    right = _build_rot(rotation)
    transform = left @ right
    cov = transform @ jnp.swapaxes(transform, -2, -1)
    chol = jnp.linalg.cholesky(cov)
    inv_cov = jax.vmap(
        lambda c: jax.scipy.linalg.cho_solve((c, True), jnp.eye(2, dtype=c.dtype))
    )(chol)
    grid_y, grid_x = jnp.meshgrid(
        jnp.arange(HW, dtype=jnp.float32),
        jnp.arange(HW, dtype=jnp.float32),
        indexing="ij",
    )
    grid = jnp.stack([grid_x, grid_y], axis=-1)
    mean = jnp.exp(mean_p)
    mean_mean = jnp.mean(mean, axis=1, keepdims=True)
    mean_std = jnp.std(mean, axis=1, keepdims=True, ddof=1)
    mean = (mean - mean_mean) / (mean_std + 1e-05)
    mean = mean * (HW // 2) + HW // 2
    mean = jnp.clip(mean, 0.0, float(HW // 2))
    diff = grid[None, :, :, :] - mean[:, None, None, :]
    maha = jnp.einsum("nhwi,nij,nhwj->nhw", diff, inv_cov, diff)
    weights = jax.nn.sigmoid(jnp.exp(-0.5 * maha)).reshape(N_W, HW * HW)
    idx = jnp.argsort(-weights, axis=1)
    sorted_w = jnp.take_along_axis(weights, idx, axis=1)

    n = jnp.arange(N_W)[:, None]
    j = jnp.arange(HW * HW)[None, :]
    wy, wx = n // W_S, n % W_S
    iy, ix = j // HW, j % HW
    ro = (wy * HW + iy) * H + wx * HW + ix
    sy, sx = idx // HW, idx % HW
    rs = (wy * HW + sy) * H + wx * HW + sx
    srow = jnp.zeros((H * H,), jnp.int32).at[ro.ravel()].set(rs.ravel().astype(jnp.int32))
    wrow = jnp.zeros((H * H,), jnp.float32).at[ro.ravel()].set(sorted_w.ravel())
    src_sp = srow + (srow >= CLS).astype(jnp.int32)
    src = jnp.concatenate(
        [src_sp[:CLS], jnp.array([CLS], jnp.int32), src_sp[CLS:]]
    )
    wgt = jnp.concatenate(
        [wrow[:CLS], jnp.array([1.0], jnp.float32), wrow[CLS:]]
    )
    src = jnp.concatenate([src, jnp.zeros((L_PAD - L,), jnp.int32)])
    wgt = jnp.concatenate([wgt, jnp.zeros((L_PAD - L,), jnp.float32)])
    return src, wgt


def _sc_body(x_hbm, src_hbm, wgt_hbm, out_hbm,
             src_v, wgt_v, wtab, in0, in1, out0, out1,
             gsem0, gsem1, ssem0, ssem1):
    info = plsc.get_sparse_core_info()
    nc = info.num_cores
    wid = lax.axis_index("s") * nc + lax.axis_index("c")
    base = wid * L

    out_b = out_hbm.at[wid]
    ins = (in0, in1)
    outs = (out0, out1)
    gsems = (gsem0, gsem1)
    ssems = (ssem0, ssem1)

    pltpu.sync_copy(src_hbm, src_v)
    pltpu.sync_copy(wgt_hbm, wgt_v)
    for k in range(L_PAD // 16):
        sl = pl.ds(k * 16, 16)
        src_v[sl] = src_v[sl] + base

    def start_gather(off, p):
        pltpu.async_copy(
            x_hbm.at[src_v.at[pl.ds(off, CHUNK)]], ins[p], gsems[p])

    def wait_gather(p):
        pltpu.make_async_copy(
            x_hbm.at[src_v.at[pl.ds(0, CHUNK)]], ins[p], gsems[p]).wait()

    def wait_scatter(p):
        pltpu.make_async_copy(
            outs[p], out_b.at[pl.ds(0, CHUNK)], ssems[p]).wait()

    def scale_rows(src_buf, dst_buf, wrow0):
        def body(r, _):
            wv = wtab[wrow0 + r, :]
            for c in range(D // 16):
                cs = pl.ds(c * 16, 16)
                dst_buf[r, cs] = src_buf[r, cs] * wv
            return 0

        lax.fori_loop(0, CHUNK, body, 0)

    start_gather(0, 0)
    start_gather(CHUNK, 1)

    def pair(g, _):
        woff = pl.multiple_of(g * 2 * CHUNK, 2 * CHUNK)
        for b in range(2):
            wch = wgt_v[pl.ds(woff + b * CHUNK, 16)]
            for r in range(16):
                wtab[b * 16 + r, :] = jnp.full((16,), wch[r], jnp.float32)
        for b in range(2):
            ch = g * 2 + b
            off = pl.multiple_of(ch * CHUNK, CHUNK)
            wait_gather(b)
            @pl.when(ch >= 2)
            def _():
                wait_scatter(b)
            scale_rows(ins[b], outs[b], b * 16)
            pltpu.async_copy(outs[b], out_b.at[pl.ds(off, CHUNK)], ssems[b])
            start_gather(pl.multiple_of(off + 2 * CHUNK, CHUNK), b)
        return 0

    lax.fori_loop(0, N_CHUNK // 2, pair, 0)

    for b in range(2):
        wait_gather(b)
        wait_scatter(b)

    toff = N_CHUNK * CHUNK
    pltpu.async_copy(
        x_hbm.at[src_v.at[pl.ds(toff, 8)]], in0.at[pl.ds(0, 8)], gsem0)
    pltpu.make_async_copy(
        x_hbm.at[src_v.at[pl.ds(0, 8)]], in0.at[pl.ds(0, 8)], gsem0).wait()
    wtail = wgt_v[pl.ds(toff, 16)]
    wvt = jnp.full((16,), wtail[0], jnp.float32)
    for c in range(D // 16):
        cs = pl.ds(c * 16, 16)
        out0[0, cs] = in0[0, cs] * wvt
    pltpu.sync_copy(out0.at[pl.ds(0, 1)], out_b.at[pl.ds(toff, 1)])


@jax.jit
def kernel(x, scale, rotation, mean):
    src, wgt = _row_tables(scale, rotation, mean)
    x2d = x.reshape(B * L, D)

    mesh = plsc.VectorSubcoreMesh(core_axis_name="c", subcore_axis_name="s")
    run = functools.partial(
        pl.kernel,
        mesh=mesh,
        out_type=jax.ShapeDtypeStruct((B, L, D), jnp.float32),
        scratch_types=[
            pltpu.VMEM((L_PAD,), jnp.int32),
            pltpu.VMEM((L_PAD,), jnp.float32),
            pltpu.VMEM((2 * CHUNK, 16), jnp.float32),
            pltpu.VMEM((CHUNK, D), jnp.float32),
            pltpu.VMEM((CHUNK, D), jnp.float32),
            pltpu.VMEM((CHUNK, D), jnp.float32),
            pltpu.VMEM((CHUNK, D), jnp.float32),
            pltpu.SemaphoreType.DMA,
            pltpu.SemaphoreType.DMA,
            pltpu.SemaphoreType.DMA,
            pltpu.SemaphoreType.DMA,
        ],
    )(_sc_body)
    return run(x2d, src, wgt)


# speedup vs baseline: 2.2318x; 1.0770x over previous
import functools
import math

import jax
import jax.numpy as jnp
from jax import lax
from jax.experimental import pallas as pl
from jax.experimental.pallas import tpu as pltpu
from jax.experimental.pallas import tpu_sc as plsc

W_S = 4
N_W = W_S * W_S
B, L, D = 32, 577, 768
CLS = L // 2
H = 24
HW = 6
L_PAD = 640
CHUNK = 16
N_CHUNK = 36


def _build_rot(r, epsilon=1e-08):
    norms = jnp.linalg.norm(r, axis=1, keepdims=True)
    r = r / (norms + epsilon)
    angles = jnp.arctan2(r[:, 0], r[:, 1])
    cos = jnp.cos(angles)
    sin = jnp.sin(angles)
    row0 = jnp.stack([cos, -sin], axis=-1)
    row1 = jnp.stack([sin, cos], axis=-1)
    return jnp.stack([row0, row1], axis=1)


def _row_tables(scale, rotation, mean_p):
    scale_e = jnp.exp(scale)
    left = jax.vmap(jnp.diag)(scale_e)
    right = _build_rot(rotation)
    transform = left @ right
    cov = transform @ jnp.swapaxes(transform, -2, -1)
    chol = jnp.linalg.cholesky(cov)
    inv_cov = jax.vmap(
        lambda c: jax.scipy.linalg.cho_solve((c, True), jnp.eye(2, dtype=c.dtype))
    )(chol)
    grid_y, grid_x = jnp.meshgrid(
        jnp.arange(HW, dtype=jnp.float32),
        jnp.arange(HW, dtype=jnp.float32),
        indexing="ij",
    )
    grid = jnp.stack([grid_x, grid_y], axis=-1)
    mean = jnp.exp(mean_p)
    mean_mean = jnp.mean(mean, axis=1, keepdims=True)
    mean_std = jnp.std(mean, axis=1, keepdims=True, ddof=1)
    mean = (mean - mean_mean) / (mean_std + 1e-05)
    mean = mean * (HW // 2) + HW // 2
    mean = jnp.clip(mean, 0.0, float(HW // 2))
    diff = grid[None, :, :, :] - mean[:, None, None, :]
    maha = jnp.einsum("nhwi,nij,nhwj->nhw", diff, inv_cov, diff)
    weights = jax.nn.sigmoid(jnp.exp(-0.5 * maha)).reshape(N_W, HW * HW)
    idx = jnp.argsort(-weights, axis=1)
    sorted_w = jnp.take_along_axis(weights, idx, axis=1)

    n = jnp.arange(N_W)[:, None]
    j = jnp.arange(HW * HW)[None, :]
    wy, wx = n // W_S, n % W_S
    iy, ix = j // HW, j % HW
    ro = (wy * HW + iy) * H + wx * HW + ix
    sy, sx = idx // HW, idx % HW
    rs = (wy * HW + sy) * H + wx * HW + sx
    srow = jnp.zeros((H * H,), jnp.int32).at[ro.ravel()].set(rs.ravel().astype(jnp.int32))
    wrow = jnp.zeros((H * H,), jnp.float32).at[ro.ravel()].set(sorted_w.ravel())
    src_sp = srow + (srow >= CLS).astype(jnp.int32)
    src = jnp.concatenate(
        [src_sp[:CLS], jnp.array([CLS], jnp.int32), src_sp[CLS:]]
    )
    wgt = jnp.concatenate(
        [wrow[:CLS], jnp.array([1.0], jnp.float32), wrow[CLS:]]
    )
    src = jnp.concatenate([src, jnp.zeros((L_PAD - L,), jnp.int32)])
    wgt = jnp.concatenate([wgt, jnp.zeros((L_PAD - L,), jnp.float32)])
    return src, wgt


def _sc_body(x_hbm, src_hbm, wgt_hbm, out_hbm,
             src_v, wgt_v, wtab, in0, in1, out0, out1,
             gsem0, gsem1, ssem0, ssem1):
    info = plsc.get_sparse_core_info()
    nc = info.num_cores
    wid = lax.axis_index("s") * nc + lax.axis_index("c")
    x_b = x_hbm.at[wid]
    out_b = out_hbm.at[wid]
    ins = (in0, in1)
    outs = (out0, out1)
    gsems = (gsem0, gsem1)
    ssems = (ssem0, ssem1)

    pltpu.sync_copy(src_hbm, src_v)
    pltpu.sync_copy(wgt_hbm, wgt_v)

    def start_gather(off, p):
        pltpu.async_copy(
            x_b.at[src_v.at[pl.ds(off, CHUNK)]], ins[p], gsems[p])

    def wait_gather(p):
        pltpu.make_async_copy(
            x_b.at[src_v.at[pl.ds(0, CHUNK)]], ins[p], gsems[p]).wait()

    def wait_scatter(p):
        pltpu.make_async_copy(
            outs[p], out_b.at[pl.ds(0, CHUNK)], ssems[p]).wait()

    def scale_rows(src_buf, dst_buf, wrow0):
        def body(r, _):
            wv = wtab[wrow0 + r, :]
            for c in range(D // 16):
                cs = pl.ds(c * 16, 16)
                dst_buf[r, cs] = src_buf[r, cs] * wv
            return 0

        lax.fori_loop(0, CHUNK, body, 0)

    start_gather(0, 0)
    start_gather(CHUNK, 1)

    def pair(g, _):
        woff = pl.multiple_of(g * 2 * CHUNK, 2 * CHUNK)
        for b in range(2):
            wch = wgt_v[pl.ds(woff + b * CHUNK, 16)]
            for r in range(16):
                wtab[b * 16 + r, :] = jnp.full((16,), wch[r], jnp.float32)
        for b in range(2):
            ch = g * 2 + b
            off = pl.multiple_of(ch * CHUNK, CHUNK)
            wait_gather(b)
            @pl.when(ch >= 2)
            def _():
                wait_scatter(b)
            scale_rows(ins[b], outs[b], b * 16)
            pltpu.async_copy(outs[b], out_b.at[pl.ds(off, CHUNK)], ssems[b])
            start_gather(pl.multiple_of(off + 2 * CHUNK, CHUNK), b)
        return 0

    lax.fori_loop(0, N_CHUNK // 2, pair, 0)

    for b in range(2):
        wait_gather(b)
        wait_scatter(b)

    toff = N_CHUNK * CHUNK
    pltpu.async_copy(
        x_b.at[src_v.at[pl.ds(toff, 8)]], in0.at[pl.ds(0, 8)], gsem0)
    pltpu.make_async_copy(
        x_b.at[src_v.at[pl.ds(0, 8)]], in0.at[pl.ds(0, 8)], gsem0).wait()
    wtail = wgt_v[pl.ds(toff, 16)]
    wvt = jnp.full((16,), wtail[0], jnp.float32)
    for c in range(D // 16):
        cs = pl.ds(c * 16, 16)
        out0[0, cs] = in0[0, cs] * wvt
    pltpu.sync_copy(out0.at[pl.ds(0, 1)], out_b.at[pl.ds(toff, 1)])


@jax.jit
def kernel(x, scale, rotation, mean):
    src, wgt = _row_tables(scale, rotation, mean)

    mesh = plsc.VectorSubcoreMesh(core_axis_name="c", subcore_axis_name="s")
    run = functools.partial(
        pl.kernel,
        mesh=mesh,
        out_type=jax.ShapeDtypeStruct((B, L, D), jnp.float32),
        scratch_types=[
            pltpu.VMEM((L_PAD,), jnp.int32),
            pltpu.VMEM((L_PAD,), jnp.float32),
            pltpu.VMEM((2 * CHUNK, 16), jnp.float32),
            pltpu.VMEM((CHUNK, D), jnp.float32),
            pltpu.VMEM((CHUNK, D), jnp.float32),
            pltpu.VMEM((CHUNK, D), jnp.float32),
            pltpu.VMEM((CHUNK, D), jnp.float32),
            pltpu.SemaphoreType.DMA,
            pltpu.SemaphoreType.DMA,
            pltpu.SemaphoreType.DMA,
            pltpu.SemaphoreType.DMA,
        ],
    )(_sc_body)
    return run(x, src, wgt)


# speedup vs baseline: 3.8519x; 1.7259x over previous
import functools
import math

import jax
import jax.numpy as jnp
from jax import lax
from jax.experimental import pallas as pl
from jax.experimental.pallas import tpu as pltpu
from jax.experimental.pallas import tpu_sc as plsc

W_S = 4
N_W = W_S * W_S
B, L, D = 32, 577, 768
CLS = L // 2
H = 24
HW = 6
NQ = 2 * L
QPT = 36
NK = 38


def _build_rot(r, epsilon=1e-08):
    norms = jnp.linalg.norm(r, axis=1, keepdims=True)
    r = r / (norms + epsilon)
    angles = jnp.arctan2(r[:, 0], r[:, 1])
    cos = jnp.cos(angles)
    sin = jnp.sin(angles)
    row0 = jnp.stack([cos, -sin], axis=-1)
    row1 = jnp.stack([sin, cos], axis=-1)
    return jnp.stack([row0, row1], axis=1)


def _row_tables(scale, rotation, mean_p):
    scale_e = jnp.exp(scale)
    left = jax.vmap(jnp.diag)(scale_e)
    right = _build_rot(rotation)
    transform = left @ right
    cov = transform @ jnp.swapaxes(transform, -2, -1)
    chol = jnp.linalg.cholesky(cov)
    inv_cov = jax.vmap(
        lambda c: jax.scipy.linalg.cho_solve((c, True), jnp.eye(2, dtype=c.dtype))
    )(chol)
    grid_y, grid_x = jnp.meshgrid(
        jnp.arange(HW, dtype=jnp.float32),
        jnp.arange(HW, dtype=jnp.float32),
        indexing="ij",
    )
    grid = jnp.stack([grid_x, grid_y], axis=-1)
    mean = jnp.exp(mean_p)
    mean_mean = jnp.mean(mean, axis=1, keepdims=True)
    mean_std = jnp.std(mean, axis=1, keepdims=True, ddof=1)
    mean = (mean - mean_mean) / (mean_std + 1e-05)
    mean = mean * (HW // 2) + HW // 2
    mean = jnp.clip(mean, 0.0, float(HW // 2))
    diff = grid[None, :, :, :] - mean[:, None, None, :]
    maha = jnp.einsum("nhwi,nij,nhwj->nhw", diff, inv_cov, diff)
    weights = jax.nn.sigmoid(jnp.exp(-0.5 * maha)).reshape(N_W, HW * HW)
    idx = jnp.argsort(-weights, axis=1)
    sorted_w = jnp.take_along_axis(weights, idx, axis=1)

    n = jnp.arange(N_W)[:, None]
    j = jnp.arange(HW * HW)[None, :]
    wy, wx = n // W_S, n % W_S
    iy, ix = j // HW, j % HW
    ro = (wy * HW + iy) * H + wx * HW + ix
    sy, sx = idx // HW, idx % HW
    rs = (wy * HW + sy) * H + wx * HW + sx
    srow = jnp.zeros((H * H,), jnp.int32).at[ro.ravel()].set(rs.ravel().astype(jnp.int32))
    wrow = jnp.zeros((H * H,), jnp.float32).at[ro.ravel()].set(sorted_w.ravel())
    src_sp = srow + (srow >= CLS).astype(jnp.int32)
    src = jnp.concatenate(
        [src_sp[:CLS], jnp.array([CLS], jnp.int32), src_sp[CLS:]]
    )
    wgt = jnp.concatenate(
        [wrow[:CLS], jnp.array([1.0], jnp.float32), wrow[CLS:]]
    )
    return src, wgt


def _tile_tables(src, wgt):
    w = jnp.arange(32)[:, None]
    k = jnp.arange(NK)[None, :]
    q = jnp.where(k == QPT, 1152 + w, w * QPT + k)
    valid = (k < QPT) | ((k == QPT) & (w < 2))
    q = jnp.where(valid, q, 0)
    p, h = q // 2, q & 1
    s2 = jnp.where(valid, src[p] * 2 + h, 0).astype(jnp.int32)
    w2 = jnp.where(valid, wgt[p], 0.0).astype(jnp.float32)
    idx8 = jnp.zeros((32, 1, 512), jnp.int32)
    idx8 = idx8.at[jnp.broadcast_to(w, (32, NK)), 0, 8 * jnp.broadcast_to(k, (32, NK))].set(s2)
    wtbl = jnp.zeros((32, 1, 128), jnp.float32)
    wtbl = wtbl.at[jnp.broadcast_to(w, (32, NK)), 0, jnp.broadcast_to(k, (32, NK))].set(w2)
    return idx8, wtbl


def _sc_body(x_hbm, idx_hbm, wtbl_hbm, out_hbm,
             idx_v, wtb_v, wtab, in0, in1, out0, out1,
             gsem0, gsem1, ssem0, ssem1):
    info = plsc.get_sparse_core_info()
    nc = info.num_cores
    wid = lax.axis_index("s") * nc + lax.axis_index("c")
    nvalid = QPT + jnp.where(wid < 2, 1, 0)

    ins = (in0, in1)
    outs = (out0, out1)
    gsems = (gsem0, gsem1)
    ssems = (ssem0, ssem1)

    pltpu.sync_copy(idx_hbm.at[wid], idx_v)
    pltpu.sync_copy(wtbl_hbm.at[wid], wtb_v)
    for j in range(3):
        wv16 = wtb_v[0, pl.ds(16 * j, 16)]
        for r in range(16):
            wtab[16 * j + r, :] = jnp.full((16,), wv16[r], jnp.float32)

    def start_gather(kk, p):
        off = pl.multiple_of(8 * kk, 8)
        pltpu.async_copy(x_hbm.at[idx_v.at[0, pl.ds(off, 1)]], ins[p], gsems[p])

    def wait_gather(p):
        pltpu.make_async_copy(
            x_hbm.at[idx_v.at[0, pl.ds(0, 1)]], ins[p], gsems[p]).wait()

    def wait_scatter(p):
        pltpu.make_async_copy(
            outs[p], out_hbm.at[pl.ds(0, 1)], ssems[p]).wait()

    def scale(src_buf, dst_buf, kk):
        wv = wtab[kk, :]

        def body(r, _):
            for c in range(D // 16):
                cs = pl.ds(c * 16, 16)
                dst_buf[0, r, cs] = src_buf[0, r, cs] * wv
            return 0

        lax.fori_loop(0, 16, body, 0)

    start_gather(0, 0)
    start_gather(1, 1)

    def pair(g, _):
        for bb in range(2):
            kk = g * 2 + bb
            wait_gather(bb)

            @pl.when((kk >= 2) & (kk - 2 < nvalid))
            def _():
                wait_scatter(bb)

            @pl.when(kk < nvalid)
            def _():
                scale(ins[bb], outs[bb], kk)
                qo = jnp.where(kk == QPT, 1152 + wid, wid * QPT + kk)
                pltpu.async_copy(outs[bb], out_hbm.at[pl.ds(qo, 1)], ssems[bb])

            start_gather(kk + 2, bb)
        return 0

    lax.fori_loop(0, NK // 2, pair, 0)

    for bb in range(2):
        wait_gather(bb)

        @pl.when(NK - 2 + bb < nvalid)
        def _():
            wait_scatter(bb)


@jax.jit
def kernel(x, scale, rotation, mean):
    src, wgt = _row_tables(scale, rotation, mean)
    idx8, wtbl = _tile_tables(src, wgt)
    x2 = jnp.transpose(x, (1, 0, 2)).reshape(NQ, 16, D)

    mesh = plsc.VectorSubcoreMesh(core_axis_name="c", subcore_axis_name="s")
    run = functools.partial(
        pl.kernel,
        mesh=mesh,
        out_type=jax.ShapeDtypeStruct((NQ, 16, D), jnp.float32),
        scratch_types=[
            pltpu.VMEM((1, 512), jnp.int32),
            pltpu.VMEM((1, 128), jnp.float32),
            pltpu.VMEM((48, 16), jnp.float32),
            pltpu.VMEM((1, 16, D), jnp.float32),
            pltpu.VMEM((1, 16, D), jnp.float32),
            pltpu.VMEM((1, 16, D), jnp.float32),
            pltpu.VMEM((1, 16, D), jnp.float32),
            pltpu.SemaphoreType.DMA,
            pltpu.SemaphoreType.DMA,
            pltpu.SemaphoreType.DMA,
            pltpu.SemaphoreType.DMA,
        ],
    )(_sc_body)
    out2 = run(x2, idx8, wtbl)
    return jnp.transpose(out2.reshape(L, B, D), (1, 0, 2))


# speedup vs baseline: 5.3671x; 1.3934x over previous
import functools
import math

import jax
import jax.numpy as jnp
from jax import lax
from jax.experimental import pallas as pl
from jax.experimental.pallas import tpu as pltpu
from jax.experimental.pallas import tpu_sc as plsc

W_S = 4
N_W = W_S * W_S
B, L, D = 32, 577, 768
CLS = L // 2
H = 24
HW = 6
NQ = 2 * L
QPT = 36
NK = 38


def _build_rot(r, epsilon=1e-08):
    norms = jnp.linalg.norm(r, axis=1, keepdims=True)
    r = r / (norms + epsilon)
    angles = jnp.arctan2(r[:, 0], r[:, 1])
    cos = jnp.cos(angles)
    sin = jnp.sin(angles)
    row0 = jnp.stack([cos, -sin], axis=-1)
    row1 = jnp.stack([sin, cos], axis=-1)
    return jnp.stack([row0, row1], axis=1)


def _row_tables(scale, rotation, mean_p):
    scale_e = jnp.exp(scale)
    left = jax.vmap(jnp.diag)(scale_e)
    right = _build_rot(rotation)
    transform = left @ right
    cov = transform @ jnp.swapaxes(transform, -2, -1)
    chol = jnp.linalg.cholesky(cov)
    inv_cov = jax.vmap(
        lambda c: jax.scipy.linalg.cho_solve((c, True), jnp.eye(2, dtype=c.dtype))
    )(chol)
    grid_y, grid_x = jnp.meshgrid(
        jnp.arange(HW, dtype=jnp.float32),
        jnp.arange(HW, dtype=jnp.float32),
        indexing="ij",
    )
    grid = jnp.stack([grid_x, grid_y], axis=-1)
    mean = jnp.exp(mean_p)
    mean_mean = jnp.mean(mean, axis=1, keepdims=True)
    mean_std = jnp.std(mean, axis=1, keepdims=True, ddof=1)
    mean = (mean - mean_mean) / (mean_std + 1e-05)
    mean = mean * (HW // 2) + HW // 2
    mean = jnp.clip(mean, 0.0, float(HW // 2))
    diff = grid[None, :, :, :] - mean[:, None, None, :]
    maha = jnp.einsum("nhwi,nij,nhwj->nhw", diff, inv_cov, diff)
    weights = jax.nn.sigmoid(jnp.exp(-0.5 * maha)).reshape(N_W, HW * HW)
    iota36 = lax.broadcasted_iota(jnp.int32, (N_W, HW * HW), 1)
    _, idx, sorted_w = lax.sort(
        (-weights, iota36, weights), dimension=1, num_keys=1, is_stable=True
    )

    n = jnp.arange(N_W)[:, None]
    wy, wx = n // W_S, n % W_S
    sy, sx = idx // HW, idx % HW
    rs = (wy * HW + sy) * H + wx * HW + sx
    srow = jnp.transpose(
        rs.astype(jnp.int32).reshape(W_S, W_S, HW, HW), (0, 2, 1, 3)
    ).reshape(H * H)
    wrow = jnp.transpose(
        sorted_w.reshape(W_S, W_S, HW, HW), (0, 2, 1, 3)
    ).reshape(H * H)
    src_sp = srow + (srow >= CLS).astype(jnp.int32)
    src = jnp.concatenate(
        [src_sp[:CLS], jnp.array([CLS], jnp.int32), src_sp[CLS:]]
    )
    wgt = jnp.concatenate(
        [wrow[:CLS], jnp.array([1.0], jnp.float32), wrow[CLS:]]
    )
    return src, wgt


def _tile_tables(src, wgt):
    s2q = (src * 2)[:, None] + jnp.arange(2, dtype=jnp.int32)[None, :]
    s2q = s2q.reshape(NQ)
    w2q = jnp.broadcast_to(wgt[:, None], (L, 2)).reshape(NQ)
    main_s = s2q[: 32 * QPT].reshape(32, QPT)
    main_w = w2q[: 32 * QPT].reshape(32, QPT)
    extra_s = jnp.concatenate(
        [s2q[32 * QPT:], jnp.zeros((30,), jnp.int32)]
    ).reshape(32, 1)
    extra_w = jnp.concatenate(
        [w2q[32 * QPT:], jnp.zeros((30,), jnp.float32)]
    ).reshape(32, 1)
    s2full = jnp.concatenate(
        [main_s, extra_s, jnp.zeros((32, 64 - QPT - 1), jnp.int32)], axis=1
    )
    wtbl = jnp.concatenate(
        [main_w, extra_w, jnp.zeros((32, 128 - QPT - 1), jnp.float32)], axis=1
    ).reshape(32, 1, 128)
    idx8 = jnp.concatenate(
        [s2full[..., None], jnp.zeros((32, 64, 7), jnp.int32)], axis=2
    ).reshape(32, 1, 512)
    return idx8, wtbl


def _sc_body(x_hbm, idx_hbm, wtbl_hbm, out_hbm,
             idx_v, wtb_v, wtab, in0, in1, out0, out1,
             gsem0, gsem1, ssem0, ssem1):
    info = plsc.get_sparse_core_info()
    nc = info.num_cores
    wid = lax.axis_index("s") * nc + lax.axis_index("c")
    nvalid = QPT + jnp.where(wid < 2, 1, 0)

    ins = (in0, in1)
    outs = (out0, out1)
    gsems = (gsem0, gsem1)
    ssems = (ssem0, ssem1)

    pltpu.sync_copy(idx_hbm.at[wid], idx_v)
    pltpu.sync_copy(wtbl_hbm.at[wid], wtb_v)
    for j in range(3):
        wv16 = wtb_v[0, pl.ds(16 * j, 16)]
        for r in range(16):
            wtab[16 * j + r, :] = jnp.full((16,), wv16[r], jnp.float32)

    def start_gather(kk, p):
        off = pl.multiple_of(8 * kk, 8)
        pltpu.async_copy(x_hbm.at[idx_v.at[0, pl.ds(off, 1)]], ins[p], gsems[p])

    def wait_gather(p):
        pltpu.make_async_copy(
            x_hbm.at[idx_v.at[0, pl.ds(0, 1)]], ins[p], gsems[p]).wait()

    def wait_scatter(p):
        pltpu.make_async_copy(
            outs[p], out_hbm.at[pl.ds(0, 1)], ssems[p]).wait()

    def scale(src_buf, dst_buf, kk):
        wv = wtab[kk, :]

        def body(r, _):
            for c in range(D // 16):
                cs = pl.ds(c * 16, 16)
                dst_buf[0, r, cs] = src_buf[0, r, cs] * wv
            return 0

        lax.fori_loop(0, 16, body, 0)

    start_gather(0, 0)
    start_gather(1, 1)

    def pair(g, _):
        for bb in range(2):
            kk = g * 2 + bb
            wait_gather(bb)

            @pl.when((kk >= 2) & (kk - 2 < nvalid))
            def _():
                wait_scatter(bb)

            @pl.when(kk < nvalid)
            def _():
                scale(ins[bb], outs[bb], kk)
                qo = jnp.where(kk == QPT, 1152 + wid, wid * QPT + kk)
                pltpu.async_copy(outs[bb], out_hbm.at[pl.ds(qo, 1)], ssems[bb])

            start_gather(kk + 2, bb)
        return 0

    lax.fori_loop(0, NK // 2, pair, 0)

    for bb in range(2):
        wait_gather(bb)

        @pl.when(NK - 2 + bb < nvalid)
        def _():
            wait_scatter(bb)


@jax.jit
def kernel(x, scale, rotation, mean):
    src, wgt = _row_tables(scale, rotation, mean)
    idx8, wtbl = _tile_tables(src, wgt)
    x2 = jnp.transpose(x, (1, 0, 2)).reshape(NQ, 16, D)

    mesh = plsc.VectorSubcoreMesh(core_axis_name="c", subcore_axis_name="s")
    run = functools.partial(
        pl.kernel,
        mesh=mesh,
        out_type=jax.ShapeDtypeStruct((NQ, 16, D), jnp.float32),
        scratch_types=[
            pltpu.VMEM((1, 512), jnp.int32),
            pltpu.VMEM((1, 128), jnp.float32),
            pltpu.VMEM((48, 16), jnp.float32),
            pltpu.VMEM((1, 16, D), jnp.float32),
            pltpu.VMEM((1, 16, D), jnp.float32),
            pltpu.VMEM((1, 16, D), jnp.float32),
            pltpu.VMEM((1, 16, D), jnp.float32),
            pltpu.SemaphoreType.DMA,
            pltpu.SemaphoreType.DMA,
            pltpu.SemaphoreType.DMA,
            pltpu.SemaphoreType.DMA,
        ],
    )(_sc_body)
    out2 = run(x2, idx8, wtbl)
    return jnp.transpose(out2.reshape(L, B, D), (1, 0, 2))
